# baseline (speedup 1.0000x reference)
import functools

import jax
import jax.numpy as jnp
from jax.experimental import pallas as pl
from jax.experimental.pallas import tpu as pltpu

B, V = 4096, 100
NUM_NAMES, NUM_ROLES, NUM_GROUPS = 100, 8, 8
NAME_D, ROLE_D, GROUP_D = 32, 16, 16
HID, TOK = 128, 64
ROWS = B * V
BLOCK_ROWS = 6400


def _ln_relu(h, g, be, eps=1e-5):
    m = jnp.mean(h, axis=1, keepdims=True)
    v = jnp.mean(h * h, axis=1, keepdims=True) - m * m
    return jnp.maximum((h - m) * jax.lax.rsqrt(v + eps) * g + be, 0.0)


def _encoder_kernel(vals_ref, nidx_ref, ridx_ref, gidx_ref,
                    ntab_ref, rtab_ref, gtab_ref,
                    w1v_ref, w1n_ref, w1r_ref, w1g_ref, b1_ref, g1_ref, be1_ref,
                    w2_ref, b2_ref, g2_ref, be2_ref,
                    w3_ref, b3_ref,
                    out_ref, base_ref):
    @pl.when(pl.program_id(0) == 0)
    def _prologue():
        ion = jax.lax.broadcasted_iota(jnp.int32, (V, NUM_NAMES), 1)
        ior = jax.lax.broadcasted_iota(jnp.int32, (V, NUM_ROLES), 1)
        iog = jax.lax.broadcasted_iota(jnp.int32, (V, NUM_GROUPS), 1)
        oh_n = (nidx_ref[...] == ion).astype(jnp.float32)
        oh_r = (ridx_ref[...] == ior).astype(jnp.float32)
        oh_g = (gidx_ref[...] == iog).astype(jnp.float32)
        dot = functools.partial(jax.lax.dot,
                                preferred_element_type=jnp.float32)
        emb_n = dot(oh_n, ntab_ref[...])
        emb_r = dot(oh_r, rtab_ref[...])
        emb_g = dot(oh_g, gtab_ref[...])
        base = (dot(emb_n, w1n_ref[...]) + dot(emb_r, w1r_ref[...])
                + dot(emb_g, w1g_ref[...]) + b1_ref[...])
        rr = jax.lax.broadcasted_iota(jnp.int32, (BLOCK_ROWS, V), 0)
        cc = jax.lax.broadcasted_iota(jnp.int32, (BLOCK_ROWS, V), 1)
        sel = (jax.lax.rem(rr, V) == cc).astype(jnp.float32)
        base_ref[...] = dot(sel, base)

    h = vals_ref[...] * w1v_ref[...] + base_ref[...]
    h = _ln_relu(h, g1_ref[...], be1_ref[...])
    h = jnp.dot(h, w2_ref[...], preferred_element_type=jnp.float32) + b2_ref[...]
    h = _ln_relu(h, g2_ref[...], be2_ref[...])
    out_ref[...] = (jnp.dot(h, w3_ref[...], preferred_element_type=jnp.float32)
                    + b3_ref[...])


def kernel(values, name_idx, role_idx, group_idx, name_table, role_table,
           group_table, W1, b1, g1, be1, W2, b2, g2, be2, W3, b3):
    vals = values.reshape(ROWS, 1)
    grid = ROWS // BLOCK_ROWS

    row_spec = pl.BlockSpec((BLOCK_ROWS, 1), lambda i: (i, 0))
    out_spec = pl.BlockSpec((BLOCK_ROWS, TOK), lambda i: (i, 0))

    def full(shape):
        return pl.BlockSpec(shape, lambda i: (0,) * len(shape))

    out = pl.pallas_call(
        _encoder_kernel,
        grid=(grid,),
        in_specs=[
            row_spec,
            full((V, 1)), full((V, 1)), full((V, 1)),
            full((NUM_NAMES, NAME_D)), full((NUM_ROLES, ROLE_D)),
            full((NUM_GROUPS, GROUP_D)),
            full((1, HID)), full((NAME_D, HID)), full((ROLE_D, HID)),
            full((GROUP_D, HID)), full((1, HID)), full((1, HID)),
            full((1, HID)),
            full((HID, HID)), full((1, HID)), full((1, HID)), full((1, HID)),
            full((HID, TOK)), full((1, TOK)),
        ],
        out_specs=out_spec,
        out_shape=jax.ShapeDtypeStruct((ROWS, TOK), jnp.float32),
        scratch_shapes=[pltpu.VMEM((BLOCK_ROWS, HID), jnp.float32)],
        compiler_params=pltpu.CompilerParams(
            dimension_semantics=("arbitrary",),
        ),
    )(
        vals,
        name_idx.reshape(V, 1), role_idx.reshape(V, 1),
        group_idx.reshape(V, 1),
        name_table, role_table, group_table,
        W1[0:1, :], W1[1:1 + NAME_D, :],
        W1[1 + NAME_D:1 + NAME_D + ROLE_D, :],
        W1[1 + NAME_D + ROLE_D:, :],
        b1.reshape(1, HID), g1.reshape(1, HID), be1.reshape(1, HID),
        W2, b2.reshape(1, HID), g2.reshape(1, HID), be2.reshape(1, HID),
        W3, b3.reshape(1, TOK),
    )
    return out.reshape(B, V, TOK)
